# baseline (speedup 1.0000x reference)
import functools
import jax
import jax.numpy as jnp
from jax import lax
from jax.experimental import pallas as pl
from jax.experimental.pallas import tpu as pltpu
from jax.experimental.pallas import tpu_sc as plsc

EMBED = 64
PAD = 128
SEQ = 200
CHUNKS = ((0, 128), (128, 72))
LANES = 16
NUM_WORKERS = 32
PAD_BLOCK = 2000


def _pad_body(w_ref, o_ref):
    o_ref[...] = jnp.pad(w_ref[...], ((0, 0), (0, PAD - EMBED)))


def _pad_table(word_table):
    vocab = word_table.shape[0]
    return pl.pallas_call(
        _pad_body,
        out_shape=jax.ShapeDtypeStruct((vocab, PAD), jnp.float32),
        grid=(vocab // PAD_BLOCK,),
        in_specs=[pl.BlockSpec((PAD_BLOCK, EMBED), lambda i: (i, 0))],
        out_specs=pl.BlockSpec((PAD_BLOCK, PAD), lambda i: (i, 0)),
    )(word_table)


def kernel(inputs, word_table, pos_table):
    batch, seq = inputs.shape
    num_idx = batch * seq
    rows_per_w = batch // NUM_WORKERS
    word_padded = _pad_table(word_table)

    mesh = plsc.VectorSubcoreMesh(core_axis_name="c", subcore_axis_name="s")

    @functools.partial(
        pl.kernel,
        out_type=jax.ShapeDtypeStruct((num_idx, EMBED), jnp.float32),
        mesh=mesh,
        scratch_types=[
            pltpu.VMEM((rows_per_w, SEQ), jnp.int32),
            pltpu.VMEM((SEQ, EMBED), jnp.float32),
            pltpu.VMEM((CHUNKS[0][1], PAD), jnp.float32),
            pltpu.VMEM((SEQ, EMBED), jnp.float32),
        ],
    )
    def sc_kernel(word_hbm, idx_hbm, pos_hbm, out_hbm,
                  idx_v, pos_v, rows_v, stage_v):
        wid = lax.axis_index("s") * 2 + lax.axis_index("c")
        row_base = pl.multiple_of(wid * rows_per_w, rows_per_w)
        pltpu.sync_copy(idx_hbm.at[pl.ds(row_base, rows_per_w)], idx_v)
        pltpu.sync_copy(pos_hbm, pos_v)

        @pl.loop(0, rows_per_w)
        def _(t):
            for start, size in CHUNKS:
                pltpu.sync_copy(
                    word_hbm.at[idx_v.at[t, pl.ds(start, size)]],
                    rows_v.at[pl.ds(0, size)],
                )

                @pl.loop(0, size)
                def _(r):
                    for c in range(0, EMBED, LANES):
                        stage_v.at[start + r, pl.ds(c, LANES)][...] = (
                            rows_v.at[r, pl.ds(c, LANES)][...]
                            + pos_v.at[start + r, pl.ds(c, LANES)][...]
                        )

            out_base = pl.multiple_of((row_base + t) * SEQ, SEQ)
            pltpu.sync_copy(stage_v, out_hbm.at[pl.ds(out_base, SEQ)])

    out = sc_kernel(word_padded, inputs, pos_table)
    return out.reshape(batch, seq, EMBED)
